# Initial kernel scaffold; baseline (speedup 1.0000x reference)
import jax
import jax.numpy as jnp
from jax.experimental import pallas as pl


def kernel(nfeats, efeats, edge_index, Wm1, bm1, Wa1, ba1, Wm2, bm2, Wa2, ba2, Wm3, bm3, Wa3, ba3, Wp, bp):
    raise NotImplementedError("write your pallas kernel here")


# speedup vs baseline: 4.8241x; 4.8241x over previous
import functools

import jax
import jax.numpy as jnp
from jax import lax
from jax.experimental import pallas as pl
from jax.experimental.pallas import tpu as pltpu
from jax.experimental.pallas import tpu_sc as plsc

N = 10000
E = 320000
NP = 10240
DIN = 128
DE = 16
DH = 152
DHP = 160
DOUT = 128
NCLS = 15

NCORES = 2
NSUB = 16
NW = NCORES * NSUB
EPT = E // NW
CH = 128
NFULL = EPT // CH
TAIL = EPT - NFULL * CH
RPT = NP // NSUB

RB = 512
GRID = NP // RB
SCW = 32

_mesh = plsc.VectorSubcoreMesh(core_axis_name="c", subcore_axis_name="s")


@functools.partial(
    pl.kernel,
    out_type=(
        jax.ShapeDtypeStruct((NCORES, NP, DIN), jnp.float32),
        jax.ShapeDtypeStruct((NCORES, NP, DE), jnp.float32),
        jax.ShapeDtypeStruct((NCORES, NP, DE), jnp.float32),
    ),
    mesh=_mesh,
    compiler_params=pltpu.CompilerParams(use_tc_tiling_on_sc=False),
    scratch_types=(
        pltpu.VMEM_SHARED((NP, DIN), jnp.float32),
        pltpu.VMEM_SHARED((NP, DE), jnp.float32),
        pltpu.VMEM_SHARED((NP, DE), jnp.float32),
        pltpu.VMEM((CH,), jnp.int32),
        pltpu.VMEM((CH,), jnp.int32),
        pltpu.VMEM((CH, DIN), jnp.float32),
        pltpu.VMEM((CH, DE), jnp.float32),
        pltpu.VMEM((CH, DE), jnp.float32),
        pltpu.VMEM((TAIL,), jnp.int32),
        pltpu.VMEM((TAIL,), jnp.int32),
        pltpu.VMEM((TAIL, DIN), jnp.float32),
        pltpu.VMEM((TAIL, DE), jnp.float32),
        pltpu.SemaphoreType.DMA,
    ),
)
def _sc_agg1(srcix, dstix, nf, ef, zh, ze, on, outh, oute, outc,
             acch, acce, accc, sidx, didx, rows, erows, ones,
             sidx_t, didx_t, rows_t, erows_t, gsem):
    cid = lax.axis_index("c")
    sid = lax.axis_index("s")
    wid = cid * NSUB + sid
    r0 = sid * RPT
    pltpu.sync_copy(zh.at[pl.ds(r0, RPT)], acch.at[pl.ds(r0, RPT)])
    pltpu.sync_copy(ze.at[pl.ds(r0, RPT)], acce.at[pl.ds(r0, RPT)])
    pltpu.sync_copy(ze.at[pl.ds(r0, RPT)], accc.at[pl.ds(r0, RPT)])
    pltpu.sync_copy(on, ones)
    plsc.subcore_barrier()

    ebase = wid * EPT

    def chunk(base, si, di, rb, eb, nrows):
        pltpu.sync_copy(srcix.at[pl.ds(base, nrows)], si)
        pltpu.sync_copy(dstix.at[pl.ds(base, nrows)], di)
        pltpu.async_copy(nf.at[si], rb, gsem).wait()
        pltpu.sync_copy(ef.at[pl.ds(base, nrows)], eb)
        pltpu.sync_copy(rb, acch.at[di], add=True)
        pltpu.sync_copy(eb, acce.at[di], add=True)
        pltpu.sync_copy(ones.at[pl.ds(0, nrows)], accc.at[di], add=True)

    @pl.loop(0, NFULL)
    def _(j):
        chunk(ebase + j * CH, sidx, didx, rows, erows, CH)

    chunk(ebase + NFULL * CH, sidx_t, didx_t, rows_t, erows_t, TAIL)

    plsc.subcore_barrier()
    pltpu.sync_copy(acch.at[pl.ds(r0, RPT)], outh.at[cid, pl.ds(r0, RPT)])
    pltpu.sync_copy(acce.at[pl.ds(r0, RPT)], oute.at[cid, pl.ds(r0, RPT)])
    pltpu.sync_copy(accc.at[pl.ds(r0, RPT)], outc.at[cid, pl.ds(r0, RPT)])


@functools.partial(
    pl.kernel,
    out_type=jax.ShapeDtypeStruct((NCORES, NP, DHP), jnp.float32),
    mesh=_mesh,
    compiler_params=pltpu.CompilerParams(use_tc_tiling_on_sc=False),
    scratch_types=(
        pltpu.VMEM_SHARED((NP, DHP), jnp.float32),
        pltpu.VMEM((CH,), jnp.int32),
        pltpu.VMEM((CH,), jnp.int32),
        pltpu.VMEM((CH, DHP), jnp.float32),
        pltpu.VMEM((TAIL,), jnp.int32),
        pltpu.VMEM((TAIL,), jnp.int32),
        pltpu.VMEM((TAIL, DHP), jnp.float32),
        pltpu.SemaphoreType.DMA,
    ),
)
def _sc_agg(srcix, dstix, h, z, out, acc, sidx, didx, rows, sidx_t, didx_t, rows_t, gsem):
    cid = lax.axis_index("c")
    sid = lax.axis_index("s")
    wid = cid * NSUB + sid
    r0 = sid * RPT
    pltpu.sync_copy(z.at[pl.ds(r0, RPT)], acc.at[pl.ds(r0, RPT)])
    plsc.subcore_barrier()

    ebase = wid * EPT

    def chunk(base, si, di, rb, nrows):
        pltpu.sync_copy(srcix.at[pl.ds(base, nrows)], si)
        pltpu.sync_copy(dstix.at[pl.ds(base, nrows)], di)
        pltpu.async_copy(h.at[si], rb, gsem).wait()
        pltpu.sync_copy(rb, acc.at[di], add=True)

    @pl.loop(0, NFULL)
    def _(j):
        chunk(ebase + j * CH, sidx, didx, rows, CH)

    chunk(ebase + NFULL * CH, sidx_t, didx_t, rows_t, TAIL)

    plsc.subcore_barrier()
    pltpu.sync_copy(acc.at[pl.ds(r0, RPT)], out.at[cid, pl.ds(r0, RPT)])


@functools.partial(
    pl.kernel,
    out_type=jax.ShapeDtypeStruct((E, DE), jnp.float32),
    mesh=_mesh,
    compiler_params=pltpu.CompilerParams(use_tc_tiling_on_sc=False),
    scratch_types=(
        pltpu.VMEM((CH,), jnp.int32),
        pltpu.VMEM((CH,), jnp.int32),
        pltpu.VMEM((CH, DE), jnp.float32),
        pltpu.VMEM((CH, DE), jnp.float32),
        pltpu.VMEM((TAIL,), jnp.int32),
        pltpu.VMEM((TAIL,), jnp.int32),
        pltpu.VMEM((TAIL, DE), jnp.float32),
        pltpu.VMEM((TAIL, DE), jnp.float32),
        pltpu.SemaphoreType.DMA,
        pltpu.SemaphoreType.DMA,
    ),
)
def _sc_pred(srcix, dstix, pu, pv, out,
             sidx, didx, abuf, bbuf, sidx_t, didx_t, abuf_t, bbuf_t, s1, s2):
    cid = lax.axis_index("c")
    sid = lax.axis_index("s")
    wid = cid * NSUB + sid
    ebase = wid * EPT

    def chunk(base, si, di, ab, bb, nrows):
        pltpu.sync_copy(srcix.at[pl.ds(base, nrows)], si)
        pltpu.sync_copy(dstix.at[pl.ds(base, nrows)], di)
        ca = pltpu.async_copy(pu.at[si], ab, s1)
        cb = pltpu.async_copy(pv.at[di], bb, s2)
        ca.wait()
        cb.wait()

        @pl.loop(0, nrows)
        def _(i):
            ab[i, :] = ab[i, :] + bb[i, :]

        pltpu.sync_copy(ab, out.at[pl.ds(base, nrows)])

    @pl.loop(0, NFULL)
    def _(j):
        chunk(ebase + j * CH, sidx, didx, abuf, bbuf, CH)

    chunk(ebase + NFULL * CH, sidx_t, didx_t, abuf_t, bbuf_t, TAIL)


def _dot(a, b):
    return jnp.dot(a, b, preferred_element_type=jnp.float32)


def _tc1_body(nf, aggh, agge, aggc, wmh, wme, bm, wah, wan, ba, h_out, sc_out):
    p = aggh[0] + aggh[1]
    se = agge[0] + agge[1]
    cv = aggc[0] + aggc[1]
    cnt = jnp.sum(cv, axis=1, keepdims=True) * (1.0 / DE)
    rinv = 1.0 / jnp.maximum(cnt, 1.0)
    neigh = (_dot(p, wmh[...]) + _dot(se, wme[...]) + bm[...][None, :] * cnt) * rinv
    h = jax.nn.relu(_dot(nf[...], wah[...]) + _dot(neigh, wan[...]) + ba[...][None, :])
    h_out[...] = jnp.concatenate(
        [h, jnp.zeros((RB, DHP - DH), jnp.float32)], axis=1)
    sc_out[...] = jnp.concatenate(
        [se, cnt, rinv, jnp.zeros((RB, SCW - DE - 2), jnp.float32)], axis=1)


def _tc_layer_body(hp, agg, sc, wmh, wme, bm, wah, wan, ba, h_out):
    p = agg[0] + agg[1]
    se = sc[:, 0:DE]
    cnt = sc[:, DE:DE + 1]
    rinv = sc[:, DE + 1:DE + 2]
    neigh = (_dot(p, wmh[...]) + _dot(se, wme[...]) + bm[...][None, :] * cnt) * rinv
    h = jax.nn.relu(_dot(hp[...], wah[...]) + _dot(neigh, wan[...]) + ba[...][None, :])
    h_out[...] = jnp.concatenate(
        [h, jnp.zeros((RB, DHP - DH), jnp.float32)], axis=1)


def _tc3_body(hp, agg, sc, wmh, wme, bm, wah, wan, ba, wpu, bp, wpv,
              pu_out, pv_out):
    p = agg[0] + agg[1]
    se = sc[:, 0:DE]
    cnt = sc[:, DE:DE + 1]
    rinv = sc[:, DE + 1:DE + 2]
    neigh = (_dot(p, wmh[...]) + _dot(se, wme[...]) + bm[...][None, :] * cnt) * rinv
    h = jax.nn.relu(_dot(hp[...], wah[...]) + _dot(neigh, wan[...]) + ba[...][None, :])
    pu_out[...] = _dot(h, wpu[...]) + bp[...][None, :]
    pv_out[...] = _dot(h, wpv[...])


def _row_spec(d):
    return pl.BlockSpec((RB, d), lambda i: (i, 0))


def _agg_spec(d):
    return pl.BlockSpec((NCORES, RB, d), lambda i: (0, i, 0))


def _full_spec(shape):
    nd = len(shape)
    return pl.BlockSpec(shape, lambda i, _nd=nd: (0,) * _nd)


def _tc1(nf, aggh, agge, aggc, wmh, wme, bm, wah, wan, ba):
    return pl.pallas_call(
        _tc1_body,
        grid=(GRID,),
        in_specs=[
            _row_spec(DIN), _agg_spec(DIN), _agg_spec(DE), _agg_spec(DE),
            _full_spec(wmh.shape), _full_spec(wme.shape), _full_spec(bm.shape),
            _full_spec(wah.shape), _full_spec(wan.shape), _full_spec(ba.shape),
        ],
        out_specs=[_row_spec(DHP), _row_spec(SCW)],
        out_shape=[
            jax.ShapeDtypeStruct((NP, DHP), jnp.float32),
            jax.ShapeDtypeStruct((NP, SCW), jnp.float32),
        ],
    )(nf, aggh, agge, aggc, wmh, wme, bm, wah, wan, ba)


def _tc_layer(hp, agg, sc, wmh, wme, bm, wah, wan, ba):
    return pl.pallas_call(
        _tc_layer_body,
        grid=(GRID,),
        in_specs=[
            _row_spec(DHP), _agg_spec(DHP), _row_spec(SCW),
            _full_spec(wmh.shape), _full_spec(wme.shape), _full_spec(bm.shape),
            _full_spec(wah.shape), _full_spec(wan.shape), _full_spec(ba.shape),
        ],
        out_specs=[_row_spec(DHP)],
        out_shape=[jax.ShapeDtypeStruct((NP, DHP), jnp.float32)],
    )(hp, agg, sc, wmh, wme, bm, wah, wan, ba)[0]


def _tc3(hp, agg, sc, wmh, wme, bm, wah, wan, ba, wpu, bp, wpv):
    return pl.pallas_call(
        _tc3_body,
        grid=(GRID,),
        in_specs=[
            _row_spec(DHP), _agg_spec(DHP), _row_spec(SCW),
            _full_spec(wmh.shape), _full_spec(wme.shape), _full_spec(bm.shape),
            _full_spec(wah.shape), _full_spec(wan.shape), _full_spec(ba.shape),
            _full_spec(wpu.shape), _full_spec(bp.shape), _full_spec(wpv.shape),
        ],
        out_specs=[_row_spec(DE), _row_spec(DE)],
        out_shape=[
            jax.ShapeDtypeStruct((NP, DE), jnp.float32),
            jax.ShapeDtypeStruct((NP, DE), jnp.float32),
        ],
    )(hp, agg, sc, wmh, wme, bm, wah, wan, ba, wpu, bp, wpv)


def _padr(w, rows):
    return jnp.pad(w, ((0, rows - w.shape[0]), (0, 0)))


def kernel(nfeats, efeats, edge_index, Wm1, bm1, Wa1, ba1, Wm2, bm2, Wa2, ba2,
           Wm3, bm3, Wa3, ba3, Wp, bp):
    nf = jnp.pad(nfeats.reshape(N, DIN), ((0, NP - N), (0, 0)))
    ef = efeats.reshape(E, DE)
    srcix = edge_index[0]
    dstix = edge_index[1]
    zh = jnp.zeros((NP, DIN), jnp.float32)
    ze = jnp.zeros((NP, DE), jnp.float32)
    z160 = jnp.zeros((NP, DHP), jnp.float32)
    on = jnp.ones((CH, DE), jnp.float32)

    aggh, agge, aggc = _sc_agg1(srcix, dstix, nf, ef, zh, ze, on)

    h1, sc = _tc1(
        nf, aggh, agge, aggc,
        Wm1[:, :DIN].T, Wm1[:, DIN:].T, bm1,
        Wa1[:, :DIN].T, Wa1[:, DIN:].T, ba1)

    agg2 = _sc_agg(srcix, dstix, h1, z160)
    h2 = _tc_layer(
        h1, agg2, sc,
        _padr(Wm2[:, :DH].T, DHP), Wm2[:, DH:].T, bm2,
        _padr(Wa2[:, :DH].T, DHP), Wa2[:, DH:].T, ba2)

    agg3 = _sc_agg(srcix, dstix, h2, z160)
    wpu = jnp.pad(Wp[:, :DOUT].T, ((0, 0), (0, DE - NCLS)))
    wpv = jnp.pad(Wp[:, DOUT:].T, ((0, 0), (0, DE - NCLS)))
    bp16 = jnp.pad(bp, (0, DE - NCLS))
    pu, pv = _tc3(
        h2, agg3, sc,
        _padr(Wm3[:, :DH].T, DHP), Wm3[:, DH:].T, bm3,
        _padr(Wa3[:, :DH].T, DHP), Wa3[:, DH:].T, ba3,
        wpu, bp16, wpv)

    score = _sc_pred(srcix, dstix, pu, pv)
    return score[:, :NCLS]


# speedup vs baseline: 6.8547x; 1.4209x over previous
import functools

import jax
import jax.numpy as jnp
from jax import lax
from jax.experimental import pallas as pl
from jax.experimental.pallas import tpu as pltpu
from jax.experimental.pallas import tpu_sc as plsc

N = 10000
E = 320000
DIN = 128
DE = 16
DEA = 32
DH = 152
DHP = 160
DHH = DHP // 2
DIH = DIN // 2
DOUT = 128
NCLS = 15

NCORES = 2
NSUB = 16
NW = NCORES * NSUB
CH = 80
NCHT = E // (NSUB * CH)
NPAIR = NCHT // 2
EPT = E // NW
NCHP = EPT // CH
RPT = N // NSUB

RB = 1000
GRID = N // RB
SCW = 32

_mesh = plsc.VectorSubcoreMesh(core_axis_name="c", subcore_axis_name="s")
_params = pltpu.CompilerParams(use_tc_tiling_on_sc=False)


def _startg(tbl, idx, buf, sem):
    pltpu.async_copy(tbl.at[idx], buf, sem)


def _waitg(tbl, idx, buf, sem):
    pltpu.make_async_copy(tbl.at[idx], buf, sem).wait()


@functools.partial(
    pl.kernel,
    out_type=(
        jax.ShapeDtypeStruct((NCORES, N, DIH), jnp.float32),
        jax.ShapeDtypeStruct((N, DEA), jnp.float32),
    ),
    mesh=_mesh,
    compiler_params=_params,
    scratch_types=(
        pltpu.VMEM_SHARED((N, DIH), jnp.float32),
        pltpu.VMEM_SHARED((N, DEA), jnp.float32),
        pltpu.VMEM((NCHT, CH), jnp.int32),
        pltpu.VMEM((NCHT, CH), jnp.int32),
        pltpu.VMEM((CH, DIH), jnp.float32),
        pltpu.VMEM((CH, DIH), jnp.float32),
        pltpu.VMEM((CH, DEA), jnp.float32),
        pltpu.VMEM((CH, DEA), jnp.float32),
        pltpu.SemaphoreType.DMA,
        pltpu.SemaphoreType.DMA,
        pltpu.SemaphoreType.DMA,
        pltpu.SemaphoreType.DMA,
    ),
)
def _sc_agg1(s2d, d2d, nfl, nfr, efa, zh, ze, outh, outec,
             acch, accec, sidx, didx, r0b, r1b, e0b, e1b, g0, g1, f0, f1):
    cid = lax.axis_index("c")
    sid = lax.axis_index("s")
    row0 = sid * RPT
    pltpu.sync_copy(zh.at[pl.ds(row0, RPT)], acch.at[pl.ds(row0, RPT)])

    @pl.when(cid == 0)
    def _():
        pltpu.sync_copy(ze.at[pl.ds(row0, RPT)], accec.at[pl.ds(row0, RPT)])

    c0 = sid * NCHT
    pltpu.sync_copy(s2d.at[pl.ds(c0, NCHT)], sidx)
    pltpu.sync_copy(d2d.at[pl.ds(c0, NCHT)], didx)
    plsc.subcore_barrier()

    def pipeline(tbl, with_ec):
        ebase = sid * NCHT * CH

        def starte(j, eb, sem):
            pltpu.async_copy(efa.at[pl.ds(ebase + j * CH, CH)], eb, sem)

        def waite(j, eb, sem):
            pltpu.make_async_copy(
                efa.at[pl.ds(ebase + j * CH, CH)], eb, sem).wait()

        def startall(j, rb, eb, gs, fs):
            _startg(tbl, sidx.at[j], rb, gs)
            if with_ec:
                starte(j, eb, fs)

        def do(j, rb, eb, gs, fs):
            _waitg(tbl, sidx.at[j], rb, gs)
            pltpu.sync_copy(rb, acch.at[didx.at[j]], add=True)
            if with_ec:
                waite(j, eb, fs)
                pltpu.sync_copy(eb, accec.at[didx.at[j]], add=True)

        startall(0, r0b, e0b, g0, f0)

        @pl.loop(0, NPAIR)
        def _(p):
            j0 = 2 * p
            startall(j0 + 1, r1b, e1b, g1, f1)
            do(j0, r0b, e0b, g0, f0)

            @pl.when(p < NPAIR - 1)
            def _():
                startall(j0 + 2, r0b, e0b, g0, f0)

            do(j0 + 1, r1b, e1b, g1, f1)

    @pl.when(cid == 0)
    def _():
        pipeline(nfl, True)

    @pl.when(cid == 1)
    def _():
        pipeline(nfr, False)

    plsc.subcore_barrier()
    pltpu.sync_copy(acch.at[pl.ds(row0, RPT)], outh.at[cid, pl.ds(row0, RPT)])

    @pl.when(cid == 0)
    def _():
        pltpu.sync_copy(accec.at[pl.ds(row0, RPT)], outec.at[pl.ds(row0, RPT)])


@functools.partial(
    pl.kernel,
    out_type=jax.ShapeDtypeStruct((NCORES, N, DHH), jnp.float32),
    mesh=_mesh,
    compiler_params=_params,
    scratch_types=(
        pltpu.VMEM_SHARED((N, DHH), jnp.float32),
        pltpu.VMEM((NCHT, CH), jnp.int32),
        pltpu.VMEM((NCHT, CH), jnp.int32),
        pltpu.VMEM((CH, DHH), jnp.float32),
        pltpu.VMEM((CH, DHH), jnp.float32),
        pltpu.SemaphoreType.DMA,
        pltpu.SemaphoreType.DMA,
    ),
)
def _sc_agg(s2d, d2d, hl, hr, z, out, acc, sidx, didx, r0b, r1b, g0, g1):
    cid = lax.axis_index("c")
    sid = lax.axis_index("s")
    row0 = sid * RPT
    pltpu.sync_copy(z.at[pl.ds(row0, RPT)], acc.at[pl.ds(row0, RPT)])
    c0 = sid * NCHT
    pltpu.sync_copy(s2d.at[pl.ds(c0, NCHT)], sidx)
    pltpu.sync_copy(d2d.at[pl.ds(c0, NCHT)], didx)
    plsc.subcore_barrier()

    def pipeline(tbl):
        def do(j, rb, gs):
            _waitg(tbl, sidx.at[j], rb, gs)
            pltpu.sync_copy(rb, acc.at[didx.at[j]], add=True)

        _startg(tbl, sidx.at[0], r0b, g0)

        @pl.loop(0, NPAIR)
        def _(p):
            j0 = 2 * p
            _startg(tbl, sidx.at[j0 + 1], r1b, g1)
            do(j0, r0b, g0)

            @pl.when(p < NPAIR - 1)
            def _():
                _startg(tbl, sidx.at[j0 + 2], r0b, g0)

            do(j0 + 1, r1b, g1)

    @pl.when(cid == 0)
    def _():
        pipeline(hl)

    @pl.when(cid == 1)
    def _():
        pipeline(hr)

    plsc.subcore_barrier()
    pltpu.sync_copy(acc.at[pl.ds(row0, RPT)], out.at[cid, pl.ds(row0, RPT)])


@functools.partial(
    pl.kernel,
    out_type=jax.ShapeDtypeStruct((E, DE), jnp.float32),
    mesh=_mesh,
    compiler_params=_params,
    scratch_types=(
        pltpu.VMEM((NCHP, CH), jnp.int32),
        pltpu.VMEM((NCHP, CH), jnp.int32),
        pltpu.VMEM((CH, DE), jnp.float32),
        pltpu.VMEM((CH, DE), jnp.float32),
        pltpu.VMEM((CH, DE), jnp.float32),
        pltpu.VMEM((CH, DE), jnp.float32),
        pltpu.VMEM((CH, DE), jnp.float32),
        pltpu.VMEM((CH, DE), jnp.float32),
        pltpu.SemaphoreType.DMA,
        pltpu.SemaphoreType.DMA,
        pltpu.SemaphoreType.DMA,
        pltpu.SemaphoreType.DMA,
        pltpu.SemaphoreType.DMA,
        pltpu.SemaphoreType.DMA,
    ),
)
def _sc_pred(s2d, d2d, pu, pv, out,
             sidx, didx, a0b, a1b, b0b, b1b, o0b, o1b,
             ga0, ga1, gb0, gb1, w0, w1):
    cid = lax.axis_index("c")
    sid = lax.axis_index("s")
    wid = cid * NSUB + sid
    c0 = wid * NCHP
    pltpu.sync_copy(s2d.at[pl.ds(c0, NCHP)], sidx)
    pltpu.sync_copy(d2d.at[pl.ds(c0, NCHP)], didx)
    ebase = wid * EPT

    def startg(j, ab, bb, gsa, gsb):
        pltpu.async_copy(pu.at[sidx.at[j]], ab, gsa)
        pltpu.async_copy(pv.at[didx.at[j]], bb, gsb)

    def dst(j):
        return out.at[pl.ds(ebase + j * CH, CH)]

    def waitw(j, ob, ws):
        pltpu.make_async_copy(ob, dst(j), ws).wait()

    def do(j, jw, ab, bb, ob, gsa, gsb, ws, first):
        pltpu.make_async_copy(pu.at[sidx.at[j]], ab, gsa).wait()
        pltpu.make_async_copy(pv.at[didx.at[j]], bb, gsb).wait()
        if not first:
            waitw(jw, ob, ws)

        @pl.loop(0, CH, unroll=8)
        def _(i):
            ob[i, :] = ab[i, :] + bb[i, :]

        pltpu.async_copy(ob, dst(j), ws)

    startg(0, a0b, b0b, ga0, gb0)
    startg(1, a1b, b1b, ga1, gb1)
    do(0, 0, a0b, b0b, o0b, ga0, gb0, w0, True)
    startg(2, a0b, b0b, ga0, gb0)
    do(1, 0, a1b, b1b, o1b, ga1, gb1, w1, True)

    @pl.loop(1, (NCHP - 1) // 2)
    def _(p):
        j0 = 2 * p
        startg(j0 + 1, a1b, b1b, ga1, gb1)
        do(j0, j0 - 2, a0b, b0b, o0b, ga0, gb0, w0, False)
        startg(j0 + 2, a0b, b0b, ga0, gb0)
        do(j0 + 1, j0 - 1, a1b, b1b, o1b, ga1, gb1, w1, False)

    jl = NCHP - 1
    do(jl, jl - 2, a0b, b0b, o0b, ga0, gb0, w0, False)
    waitw(jl - 1, o1b, w1)
    waitw(jl, o0b, w0)


def _dot(a, b):
    return jnp.dot(a, b, preferred_element_type=jnp.float32)


def _relu_layer(p, se, cnt, rinv, hp, wmh, wme, bm, wah, wan, ba):
    neigh = (_dot(p, wmh[...]) + _dot(se, wme[...])
             + bm[...][None, :] * cnt) * rinv
    return jax.nn.relu(
        _dot(hp, wah[...]) + _dot(neigh, wan[...]) + ba[...][None, :])


def _split_h(h, hl_out, hr_out):
    hl_out[...] = h[:, :DHH]
    hr_out[...] = jnp.concatenate(
        [h[:, DHH:], jnp.zeros((RB, DHP - DH), jnp.float32)], axis=1)


def _tc1_body(nfl, nfr, aggh, aggec, wmh, wme, bm, wah, wan, ba,
              hl_out, hr_out, sc_out):
    p = jnp.concatenate([aggh[0], aggh[1]], axis=1)
    se = aggec[:, 0:DE]
    cnt = aggec[:, DE:DE + 1]
    rinv = 1.0 / jnp.maximum(cnt, 1.0)
    nf = jnp.concatenate([nfl[...], nfr[...]], axis=1)
    h = _relu_layer(p, se, cnt, rinv, nf, wmh, wme, bm, wah, wan, ba)
    _split_h(h, hl_out, hr_out)
    sc_out[...] = jnp.concatenate(
        [se, cnt, rinv, jnp.zeros((RB, SCW - DE - 2), jnp.float32)], axis=1)


def _tc_layer_body(hl, hr, agg, sc, wmh, wme, bm, wah, wan, ba,
                   hl_out, hr_out):
    p = jnp.concatenate([agg[0], agg[1]], axis=1)
    se = sc[:, 0:DE]
    cnt = sc[:, DE:DE + 1]
    rinv = sc[:, DE + 1:DE + 2]
    hp = jnp.concatenate([hl[...], hr[...]], axis=1)
    h = _relu_layer(p, se, cnt, rinv, hp, wmh, wme, bm, wah, wan, ba)
    _split_h(h, hl_out, hr_out)


def _tc3_body(hl, hr, agg, sc, wmh, wme, bm, wah, wan, ba, wpu, bp, wpv,
              pu_out, pv_out):
    p = jnp.concatenate([agg[0], agg[1]], axis=1)
    se = sc[:, 0:DE]
    cnt = sc[:, DE:DE + 1]
    rinv = sc[:, DE + 1:DE + 2]
    hp = jnp.concatenate([hl[...], hr[...]], axis=1)
    h = _relu_layer(p, se, cnt, rinv, hp, wmh, wme, bm, wah, wan, ba)
    pu_out[...] = _dot(h, wpu[...]) + bp[...][None, :]
    pv_out[...] = _dot(h, wpv[...])


def _row_spec(d):
    return pl.BlockSpec((RB, d), lambda i: (i, 0))


def _agg_spec(d):
    return pl.BlockSpec((NCORES, RB, d), lambda i: (0, i, 0))


def _full_spec(shape):
    nd = len(shape)
    return pl.BlockSpec(shape, lambda i, _nd=nd: (0,) * _nd)


def _tc1(nfl, nfr, aggh, aggec, wmh, wme, bm, wah, wan, ba):
    return pl.pallas_call(
        _tc1_body,
        grid=(GRID,),
        in_specs=[
            _row_spec(DIH), _row_spec(DIH), _agg_spec(DIH), _row_spec(DEA),
            _full_spec(wmh.shape), _full_spec(wme.shape), _full_spec(bm.shape),
            _full_spec(wah.shape), _full_spec(wan.shape), _full_spec(ba.shape),
        ],
        out_specs=[_row_spec(DHH), _row_spec(DHH), _row_spec(SCW)],
        out_shape=[
            jax.ShapeDtypeStruct((N, DHH), jnp.float32),
            jax.ShapeDtypeStruct((N, DHH), jnp.float32),
            jax.ShapeDtypeStruct((N, SCW), jnp.float32),
        ],
    )(nfl, nfr, aggh, aggec, wmh, wme, bm, wah, wan, ba)


def _tc_layer(hl, hr, agg, sc, wmh, wme, bm, wah, wan, ba):
    return pl.pallas_call(
        _tc_layer_body,
        grid=(GRID,),
        in_specs=[
            _row_spec(DHH), _row_spec(DHH), _agg_spec(DHH), _row_spec(SCW),
            _full_spec(wmh.shape), _full_spec(wme.shape), _full_spec(bm.shape),
            _full_spec(wah.shape), _full_spec(wan.shape), _full_spec(ba.shape),
        ],
        out_specs=[_row_spec(DHH), _row_spec(DHH)],
        out_shape=[
            jax.ShapeDtypeStruct((N, DHH), jnp.float32),
            jax.ShapeDtypeStruct((N, DHH), jnp.float32),
        ],
    )(hl, hr, agg, sc, wmh, wme, bm, wah, wan, ba)


def _tc3(hl, hr, agg, sc, wmh, wme, bm, wah, wan, ba, wpu, bp, wpv):
    return pl.pallas_call(
        _tc3_body,
        grid=(GRID,),
        in_specs=[
            _row_spec(DHH), _row_spec(DHH), _agg_spec(DHH), _row_spec(SCW),
            _full_spec(wmh.shape), _full_spec(wme.shape), _full_spec(bm.shape),
            _full_spec(wah.shape), _full_spec(wan.shape), _full_spec(ba.shape),
            _full_spec(wpu.shape), _full_spec(bp.shape), _full_spec(wpv.shape),
        ],
        out_specs=[_row_spec(DE), _row_spec(DE)],
        out_shape=[
            jax.ShapeDtypeStruct((N, DE), jnp.float32),
            jax.ShapeDtypeStruct((N, DE), jnp.float32),
        ],
    )(hl, hr, agg, sc, wmh, wme, bm, wah, wan, ba, wpu, bp, wpv)


def _padr(w, rows):
    return jnp.pad(w, ((0, rows - w.shape[0]), (0, 0)))


def kernel(nfeats, efeats, edge_index, Wm1, bm1, Wa1, ba1, Wm2, bm2, Wa2, ba2,
           Wm3, bm3, Wa3, ba3, Wp, bp):
    nf = nfeats.reshape(N, DIN)
    nfl = nf[:, :DIH]
    nfr = nf[:, DIH:]
    ef = efeats.reshape(E, DE)
    efa = jnp.concatenate(
        [ef, jnp.ones((E, 1), jnp.float32),
         jnp.zeros((E, DEA - DE - 1), jnp.float32)], axis=1)
    s2d = edge_index[0].reshape(E // CH, CH)
    d2d = edge_index[1].reshape(E // CH, CH)
    zh = jnp.zeros((N, DIH), jnp.float32)
    ze = jnp.zeros((N, DEA), jnp.float32)
    z80 = jnp.zeros((N, DHH), jnp.float32)

    aggh, aggec = _sc_agg1(s2d, d2d, nfl, nfr, efa, zh, ze)

    h1l, h1r, sc = _tc1(
        nfl, nfr, aggh, aggec,
        Wm1[:, :DIN].T, Wm1[:, DIN:].T, bm1,
        Wa1[:, :DIN].T, Wa1[:, DIN:].T, ba1)

    agg2 = _sc_agg(s2d, d2d, h1l, h1r, z80)
    h2l, h2r = _tc_layer(
        h1l, h1r, agg2, sc,
        _padr(Wm2[:, :DH].T, DHP), Wm2[:, DH:].T, bm2,
        _padr(Wa2[:, :DH].T, DHP), Wa2[:, DH:].T, ba2)

    agg3 = _sc_agg(s2d, d2d, h2l, h2r, z80)
    wpu = jnp.pad(Wp[:, :DOUT].T, ((0, 0), (0, DE - NCLS)))
    wpv = jnp.pad(Wp[:, DOUT:].T, ((0, 0), (0, DE - NCLS)))
    bp16 = jnp.pad(bp, (0, DE - NCLS))
    pu, pv = _tc3(
        h2l, h2r, agg3, sc,
        _padr(Wm3[:, :DH].T, DHP), Wm3[:, DH:].T, bm3,
        _padr(Wa3[:, :DH].T, DHP), Wa3[:, DH:].T, ba3,
        wpu, bp16, wpv)

    score = _sc_pred(s2d, d2d, pu, pv)
    return score[:, :NCLS]
